# baseline (speedup 1.0000x reference)
import functools

import jax
import jax.numpy as jnp
from jax import lax
from jax.experimental import pallas as pl
from jax.experimental.pallas import tpu as pltpu
from jax.experimental.pallas import tpu_sc as plsc

_N_VOX = 10000
_N_PTS = 320000
_D = 128
_B = 16
_SCALE = 14.285714285714286

_NC = 2
_NS = 16
_L = 16
_NW = _NC * _NS
_P = _N_PTS // _NW
_NV = _P // _L
_CW = 128
_VR = 80
_NVP = _VR * _CW
_U = 2
_LR = _U * _VR
_HR = _B * _VR


def _sc_hist_body(v2p_hbm, bidx_hbm, out_hbm, v2p_v, b_v, hist_v, idx_v,
                  hist_sh):
    cid = lax.axis_index("c")
    sid = lax.axis_index("s")
    wid = sid * _NC + cid
    base = wid * _P

    pltpu.sync_copy(v2p_hbm.at[pl.ds(base, _P)], v2p_v)
    pltpu.sync_copy(bidx_hbm.at[pl.ds(base, _P)], b_v)

    zeros = jnp.zeros((_L,), jnp.float32)

    def zero_hist():
        @plsc.parallel_loop(0, _LR, unroll=8)
        def _(r):
            for c in range(_CW // _L):
                hist_v[r, pl.ds(c * _L, _L)] = zeros

    zero_hist()
    pltpu.sync_copy(hist_v.at[pl.ds(0, _VR)],
                    hist_sh.at[pl.ds(sid * _VR, _VR)])
    plsc.subcore_barrier()

    b_lo = b_v[pl.ds(0, _L)][0]
    b_hi = b_v[pl.ds(_P - _L, _L)][_L - 1]
    lanes = lax.iota(jnp.int32, _L)

    def batch_pass(bb, carry):
        @plsc.parallel_loop(0, _NV, unroll=8)
        def _(g):
            sl = pl.ds(g * _L, _L)
            v = v2p_v[sl]
            m = b_v[sl] == bb
            cnt, last = plsc.scan_count(v, mask=m)
            sub = (g & (_U - 1)) * _VR
            plsc.addupdate_scatter(
                hist_v, [sub + (v >> 7), v & 127], cnt.astype(jnp.float32),
                mask=last)

        row0 = bb * _VR
        for c in range(_LR // _L):
            idx_v[pl.ds(c * _L, _L)] = row0 + (c % (_VR // _L)) * _L + lanes
        pltpu.sync_copy(hist_v, hist_sh.at[idx_v], add=True)
        zero_hist()
        return carry

    lax.fori_loop(b_lo, b_hi + 1, batch_pass, 0)
    plsc.subcore_barrier()

    pltpu.sync_copy(hist_sh.at[pl.ds(sid * _VR, _VR)],
                    out_hbm.at[cid, pl.ds(sid * _VR, _VR)])


@functools.cache
def _sc_hist_kernel():
    mesh = plsc.VectorSubcoreMesh(
        core_axis_name="c", subcore_axis_name="s", num_cores=_NC,
        num_subcores=_NS)
    return pl.kernel(
        _sc_hist_body,
        out_type=jax.ShapeDtypeStruct((_NC, _HR, _CW), jnp.float32),
        mesh=mesh,
        compiler_params=pltpu.CompilerParams(needs_layout_passes=False,
                                             skip_device_barrier=True),
        scratch_types=[
            pltpu.VMEM((_P,), jnp.int32),
            pltpu.VMEM((_P,), jnp.int32),
            pltpu.VMEM((_LR, _CW), jnp.float32),
            pltpu.VMEM((_LR,), jnp.int32),
            pltpu.VMEM_SHARED((_HR, _CW), jnp.float32),
        ],
    )


def _tc_finish_body(hist_ref, feats_ref, cap_ref, out_ref, lab_ref):
    h = (hist_ref[0] + hist_ref[1])[:, :_N_VOX]
    counts = jnp.sum(h, axis=1, keepdims=True)
    sums = jnp.dot(h, feats_ref[...], preferred_element_type=jnp.float32)
    pooled = sums / jnp.maximum(counts, 1.0)
    sq = jnp.sum(pooled * pooled, axis=1, keepdims=True)
    pooled_n = pooled / jnp.maximum(jnp.sqrt(sq), 1e-12)
    logits = jnp.dot(pooled_n, cap_ref[...].T,
                     preferred_element_type=jnp.float32)
    out_ref[...] = logits * _SCALE
    lab_ref[...] = lax.broadcasted_iota(jnp.int32, (_B,), 0)


@functools.cache
def _tc_finish_kernel():
    return pl.pallas_call(
        _tc_finish_body,
        out_shape=(jax.ShapeDtypeStruct((_B, _B), jnp.float32),
                   jax.ShapeDtypeStruct((_B,), jnp.int32)),
        compiler_params=pltpu.CompilerParams(skip_device_barrier=True),
    )


def kernel(adapter_feats, caption_embed, v2p_map, batch_idx):
    v2p = v2p_map.astype(jnp.int32)
    bid = batch_idx.astype(jnp.int32)
    hist = _sc_hist_kernel()(v2p, bid)
    hist = hist.reshape(_NC, _B, _NVP)
    caption_logit, caption_labels = _tc_finish_kernel()(
        hist, adapter_feats, caption_embed)
    caption_labels = caption_labels.astype(jnp.int64)
    return caption_logit, caption_labels


# speedup vs baseline: 1.0144x; 1.0144x over previous
import functools

import jax
import jax.numpy as jnp
from jax import lax
from jax.experimental import pallas as pl
from jax.experimental.pallas import tpu as pltpu
from jax.experimental.pallas import tpu_sc as plsc

_N_VOX = 10000
_N_PTS = 320000
_D = 128
_B = 16
_SCALE = 14.285714285714286

_NC = 2
_NS = 16
_L = 16
_NW = _NC * _NS
_P = _N_PTS // _NW
_NV = _P // _L
_CW = 128
_VR = 80
_NVP = _VR * _CW
_U = 2
_LR = _U * _VR
_HR = _B * _VR


def _sc_hist_body(v2p_hbm, bidx_hbm, out_hbm, v2p_v, b_v, hist_v, idx_v,
                  hist_sh):
    cid = lax.axis_index("c")
    sid = lax.axis_index("s")
    wid = sid * _NC + cid
    base = wid * _P

    pltpu.sync_copy(v2p_hbm.at[pl.ds(base, _P)], v2p_v)
    pltpu.sync_copy(bidx_hbm.at[pl.ds(base, _P)], b_v)

    zeros = jnp.zeros((_L,), jnp.float32)

    def zero_hist():
        @plsc.parallel_loop(0, _LR, unroll=4)
        def _(r):
            for c in range(_CW // _L):
                hist_v[r, pl.ds(c * _L, _L)] = zeros

    zero_hist()
    pltpu.sync_copy(hist_v.at[pl.ds(0, _VR)],
                    hist_sh.at[pl.ds(sid * _VR, _VR)])
    plsc.subcore_barrier()

    b_lo = b_v[pl.ds(0, _L)][0]
    b_hi = b_v[pl.ds(_P - _L, _L)][_L - 1]
    lanes = lax.iota(jnp.int32, _L)

    def batch_pass(bb, carry):
        @plsc.parallel_loop(0, _NV, unroll=4)
        def _(g):
            sl = pl.ds(g * _L, _L)
            v = v2p_v[sl]
            m = b_v[sl] == bb
            cnt, last = plsc.scan_count(v, mask=m)
            sub = (g & (_U - 1)) * _VR
            plsc.addupdate_scatter(
                hist_v, [sub + (v >> 7), v & 127], cnt.astype(jnp.float32),
                mask=last)

        row0 = bb * _VR
        for c in range(_LR // _L):
            idx_v[pl.ds(c * _L, _L)] = row0 + (c % (_VR // _L)) * _L + lanes
        pltpu.sync_copy(hist_v, hist_sh.at[idx_v], add=True)
        zero_hist()
        return carry

    lax.fori_loop(b_lo, b_hi + 1, batch_pass, 0)
    plsc.subcore_barrier()

    pltpu.sync_copy(hist_sh.at[pl.ds(sid * _VR, _VR)],
                    out_hbm.at[cid, pl.ds(sid * _VR, _VR)])


@functools.cache
def _sc_hist_kernel():
    mesh = plsc.VectorSubcoreMesh(
        core_axis_name="c", subcore_axis_name="s", num_cores=_NC,
        num_subcores=_NS)
    return pl.kernel(
        _sc_hist_body,
        out_type=jax.ShapeDtypeStruct((_NC, _HR, _CW), jnp.float32),
        mesh=mesh,
        compiler_params=pltpu.CompilerParams(needs_layout_passes=False,
                                             skip_device_barrier=True),
        scratch_types=[
            pltpu.VMEM((_P,), jnp.int32),
            pltpu.VMEM((_P,), jnp.int32),
            pltpu.VMEM((_LR, _CW), jnp.float32),
            pltpu.VMEM((_LR,), jnp.int32),
            pltpu.VMEM_SHARED((_HR, _CW), jnp.float32),
        ],
    )


def _tc_finish_body(hist_ref, feats_ref, cap_ref, out_ref, lab_ref):
    h = (hist_ref[0] + hist_ref[1])[:, :_N_VOX]
    counts = jnp.sum(h, axis=1, keepdims=True)
    sums = jnp.dot(h, feats_ref[...], preferred_element_type=jnp.float32)
    pooled = sums / jnp.maximum(counts, 1.0)
    sq = jnp.sum(pooled * pooled, axis=1, keepdims=True)
    pooled_n = pooled / jnp.maximum(jnp.sqrt(sq), 1e-12)
    logits = jnp.dot(pooled_n, cap_ref[...].T,
                     preferred_element_type=jnp.float32)
    out_ref[...] = logits * _SCALE
    lab_ref[...] = lax.broadcasted_iota(jnp.int32, (_B,), 0)


@functools.cache
def _tc_finish_kernel():
    return pl.pallas_call(
        _tc_finish_body,
        out_shape=(jax.ShapeDtypeStruct((_B, _B), jnp.float32),
                   jax.ShapeDtypeStruct((_B,), jnp.int32)),
        compiler_params=pltpu.CompilerParams(skip_device_barrier=True),
    )


def kernel(adapter_feats, caption_embed, v2p_map, batch_idx):
    v2p = v2p_map.astype(jnp.int32)
    bid = batch_idx.astype(jnp.int32)
    hist = _sc_hist_kernel()(v2p, bid)
    hist = hist.reshape(_NC, _B, _NVP)
    caption_logit, caption_labels = _tc_finish_kernel()(
        hist, adapter_feats, caption_embed)
    caption_labels = caption_labels.astype(jnp.int64)
    return caption_logit, caption_labels
